# Initial kernel scaffold; baseline (speedup 1.0000x reference)
import jax
import jax.numpy as jnp
from jax.experimental import pallas as pl


def kernel(sequence, table):
    raise NotImplementedError("write your pallas kernel here")


# speedup vs baseline: 23.1382x; 23.1382x over previous
import jax
import jax.numpy as jnp
from jax.experimental import pallas as pl

_BB = 128


def _body(tab_ref, out_ref):
    out_ref[...] = jnp.broadcast_to(tab_ref[...][None, :, :], out_ref.shape)


def kernel(sequence, table):
    b, l = sequence.shape
    d = table.shape[1]
    return pl.pallas_call(
        _body,
        grid=(b // _BB,),
        in_specs=[pl.BlockSpec((l, d), lambda i: (0, 0))],
        out_specs=pl.BlockSpec((_BB, l, d), lambda i: (i, 0, 0)),
        out_shape=jax.ShapeDtypeStruct((b, l, d), table.dtype),
    )(table)
